# Initial kernel scaffold; baseline (speedup 1.0000x reference)
import jax
import jax.numpy as jnp
from jax.experimental import pallas as pl


def kernel(phase_indices, mag_indices, phase_cos_table, mag_exp_table):
    raise NotImplementedError("write your pallas kernel here")


# speedup vs baseline: 486.6250x; 486.6250x over previous
import functools

import jax
import jax.numpy as jnp
from jax import lax
from jax.experimental import pallas as pl
from jax.experimental.pallas import tpu as pltpu
from jax.experimental.pallas import tpu_sc as plsc

D = 8388608
N_PH = 64
N_MG = 1024

NC = 2
NS = 16
L = 16
NW = NC * NS
PER_W = D // NW
CHUNK = 8192
N_CHUNKS = PER_W // CHUNK
UNROLL = 8
GROUPS = CHUNK // (L * UNROLL)

_mesh = plsc.VectorSubcoreMesh(core_axis_name="c", subcore_axis_name="s")


@functools.partial(
    pl.kernel,
    mesh=_mesh,
    out_type=jax.ShapeDtypeStruct((D,), jnp.float32),
    compiler_params=pltpu.CompilerParams(
        needs_layout_passes=False, use_tc_tiling_on_sc=False),
    scratch_types=[
        pltpu.VMEM((N_PH,), jnp.float32),
        pltpu.VMEM((N_MG,), jnp.float32),
        pltpu.VMEM((CHUNK,), jnp.int32),
        pltpu.VMEM((CHUNK,), jnp.int32),
        pltpu.VMEM((CHUNK,), jnp.int32),
        pltpu.VMEM((CHUNK,), jnp.int32),
        pltpu.VMEM((CHUNK,), jnp.float32),
        pltpu.VMEM((CHUNK,), jnp.float32),
        pltpu.SemaphoreType.DMA,
        pltpu.SemaphoreType.DMA,
        pltpu.SemaphoreType.DMA,
        pltpu.SemaphoreType.DMA,
    ],
)
def _sc_lookup(pi_hbm, mi_hbm, pct_hbm, met_hbm, out_hbm,
               pct_v, met_v, pi0, pi1, mi0, mi1, o0, o1,
               sem_in0, sem_in1, sem_out0, sem_out1):
    wid = lax.axis_index("s") * NC + lax.axis_index("c")
    base = wid * PER_W

    pi_bufs = (pi0, pi1)
    mi_bufs = (mi0, mi1)
    o_bufs = (o0, o1)
    sems_in = (sem_in0, sem_in1)
    sems_out = (sem_out0, sem_out1)

    pltpu.sync_copy(pct_hbm, pct_v)
    pltpu.sync_copy(met_hbm, met_v)

    def start_in(g, b):
        off = base + g * CHUNK
        pltpu.async_copy(pi_hbm.at[pl.ds(off, CHUNK)], pi_bufs[b], sems_in[b])
        pltpu.async_copy(mi_hbm.at[pl.ds(off, CHUNK)], mi_bufs[b], sems_in[b])

    def wait_in(b):
        pltpu.make_async_copy(pi_hbm.at[pl.ds(0, CHUNK)], pi_bufs[b],
                              sems_in[b]).wait()
        pltpu.make_async_copy(mi_hbm.at[pl.ds(0, CHUNK)], mi_bufs[b],
                              sems_in[b]).wait()

    def start_out(g, b):
        off = base + g * CHUNK
        pltpu.async_copy(o_bufs[b], out_hbm.at[pl.ds(off, CHUNK)], sems_out[b])

    def wait_out(b):
        pltpu.make_async_copy(o_bufs[b], out_hbm.at[pl.ds(0, CHUNK)],
                              sems_out[b]).wait()

    def compute(b):
        pi_buf, mi_buf, o_buf = pi_bufs[b], mi_bufs[b], o_bufs[b]

        def body(i, _):
            ibase = i * (L * UNROLL)
            for j in range(UNROLL):
                off = ibase + j * L
                pidx = pi_buf[pl.ds(off, L)]
                midx = mi_buf[pl.ds(off, L)]
                pidx = lax.max(jnp.int32(0), lax.min(pidx, jnp.int32(N_PH - 1)))
                midx = lax.max(jnp.int32(0), lax.min(midx, jnp.int32(N_MG - 1)))
                cv = plsc.load_gather(pct_v, [pidx])
                mv = plsc.load_gather(met_v, [midx])
                o_buf[pl.ds(off, L)] = cv * mv
            return 0

        lax.fori_loop(0, GROUPS, body, 0)

    start_in(0, 0)
    start_in(1, 1)
    for g in range(N_CHUNKS):
        b = g & 1
        wait_in(b)
        if g >= 2:
            wait_out(b)
        compute(b)
        start_out(g, b)
        if g + 2 < N_CHUNKS:
            start_in(g + 2, b)
    wait_out(0)
    wait_out(1)


def kernel(phase_indices, mag_indices, phase_cos_table, mag_exp_table):
    pi = phase_indices.astype(jnp.int32)
    mi = mag_indices.astype(jnp.int32)
    pct = phase_cos_table.astype(jnp.float32)
    met = mag_exp_table.astype(jnp.float32)
    return _sc_lookup(pi, mi, pct, met)


# speedup vs baseline: 504.2030x; 1.0361x over previous
import functools

import jax
import jax.numpy as jnp
from jax import lax
from jax.experimental import pallas as pl
from jax.experimental.pallas import tpu as pltpu
from jax.experimental.pallas import tpu_sc as plsc

D = 8388608
N_PH = 64
N_MG = 1024

NC = 2
NS = 16
L = 16
NW = NC * NS
PER_W = D // NW
CHUNK = 8192
N_CHUNKS = PER_W // CHUNK
UNROLL = 8
GROUPS = CHUNK // (L * UNROLL)

_mesh = plsc.VectorSubcoreMesh(core_axis_name="c", subcore_axis_name="s")


@functools.partial(
    pl.kernel,
    mesh=_mesh,
    out_type=jax.ShapeDtypeStruct((D,), jnp.float32),
    compiler_params=pltpu.CompilerParams(
        needs_layout_passes=False, use_tc_tiling_on_sc=False),
    scratch_types=[
        pltpu.VMEM((N_PH,), jnp.float32),
        pltpu.VMEM((N_MG,), jnp.float32),
        pltpu.VMEM((N_PH * L,), jnp.float32),
        pltpu.VMEM((N_MG * L,), jnp.float32),
        pltpu.VMEM((CHUNK,), jnp.int32),
        pltpu.VMEM((CHUNK,), jnp.int32),
        pltpu.VMEM((CHUNK,), jnp.int32),
        pltpu.VMEM((CHUNK,), jnp.int32),
        pltpu.VMEM((CHUNK,), jnp.float32),
        pltpu.VMEM((CHUNK,), jnp.float32),
        pltpu.SemaphoreType.DMA,
        pltpu.SemaphoreType.DMA,
        pltpu.SemaphoreType.DMA,
        pltpu.SemaphoreType.DMA,
    ],
)
def _sc_lookup(pi_hbm, mi_hbm, pct_hbm, met_hbm, out_hbm,
               pct_v, met_v, pct_rep, met_rep, pi0, pi1, mi0, mi1, o0, o1,
               sem_in0, sem_in1, sem_out0, sem_out1):
    wid = lax.axis_index("s") * NC + lax.axis_index("c")
    base = wid * PER_W

    pi_bufs = (pi0, pi1)
    mi_bufs = (mi0, mi1)
    o_bufs = (o0, o1)
    sems_in = (sem_in0, sem_in1)
    sems_out = (sem_out0, sem_out1)

    pltpu.sync_copy(pct_hbm, pct_v)
    pltpu.sync_copy(met_hbm, met_v)

    lanes = lax.iota(jnp.int32, L)

    def build_rep(src_ref, rep_ref, n):
        def body(i, _):
            v = plsc.load_gather(src_ref, [jnp.full((L,), i, jnp.int32)])
            rep_ref[pl.ds(i * L, L)] = v
            return 0
        lax.fori_loop(0, n, body, 0)

    build_rep(pct_v, pct_rep, N_PH)
    build_rep(met_v, met_rep, N_MG)

    def start_in(g, b):
        off = base + g * CHUNK
        pltpu.async_copy(pi_hbm.at[pl.ds(off, CHUNK)], pi_bufs[b], sems_in[b])
        pltpu.async_copy(mi_hbm.at[pl.ds(off, CHUNK)], mi_bufs[b], sems_in[b])

    def wait_in(b):
        pltpu.make_async_copy(pi_hbm.at[pl.ds(0, CHUNK)], pi_bufs[b],
                              sems_in[b]).wait()
        pltpu.make_async_copy(mi_hbm.at[pl.ds(0, CHUNK)], mi_bufs[b],
                              sems_in[b]).wait()

    def start_out(g, b):
        off = base + g * CHUNK
        pltpu.async_copy(o_bufs[b], out_hbm.at[pl.ds(off, CHUNK)], sems_out[b])

    def wait_out(b):
        pltpu.make_async_copy(o_bufs[b], out_hbm.at[pl.ds(0, CHUNK)],
                              sems_out[b]).wait()

    def compute(b):
        pi_buf, mi_buf, o_buf = pi_bufs[b], mi_bufs[b], o_bufs[b]

        def body(i, _):
            ibase = i * (L * UNROLL)
            for j in range(UNROLL):
                off = ibase + j * L
                pidx = pi_buf[pl.ds(off, L)]
                midx = mi_buf[pl.ds(off, L)]
                pidx = lax.max(jnp.int32(0), lax.min(pidx, jnp.int32(N_PH - 1)))
                midx = lax.max(jnp.int32(0), lax.min(midx, jnp.int32(N_MG - 1)))
                cv = plsc.load_gather(pct_rep, [(pidx << 4) | lanes])
                mv = plsc.load_gather(met_rep, [(midx << 4) | lanes])
                o_buf[pl.ds(off, L)] = cv * mv
            return 0

        lax.fori_loop(0, GROUPS, body, 0)

    start_in(0, 0)
    start_in(1, 1)
    for g in range(N_CHUNKS):
        b = g & 1
        wait_in(b)
        if g >= 2:
            wait_out(b)
        compute(b)
        start_out(g, b)
        if g + 2 < N_CHUNKS:
            start_in(g + 2, b)
    wait_out(0)
    wait_out(1)


def kernel(phase_indices, mag_indices, phase_cos_table, mag_exp_table):
    pi = phase_indices.astype(jnp.int32)
    mi = mag_indices.astype(jnp.int32)
    pct = phase_cos_table.astype(jnp.float32)
    met = mag_exp_table.astype(jnp.float32)
    return _sc_lookup(pi, mi, pct, met)


# speedup vs baseline: 825.3306x; 1.6369x over previous
import functools

import jax
import jax.numpy as jnp
from jax import lax
from jax.experimental import pallas as pl
from jax.experimental.pallas import tpu as pltpu
from jax.experimental.pallas import tpu_sc as plsc

D = 8388608
N_PH = 64
N_MG = 1024

NC = 2
NS = 16
L = 16
NW = NC * NS
PER_W = D // NW
CHUNK = 8192
N_CHUNKS = PER_W // CHUNK
UNROLL = 8
GROUPS = CHUNK // (L * UNROLL)

_mesh = plsc.VectorSubcoreMesh(core_axis_name="c", subcore_axis_name="s")


@functools.partial(
    pl.kernel,
    mesh=_mesh,
    out_type=jax.ShapeDtypeStruct((D,), jnp.float32),
    compiler_params=pltpu.CompilerParams(
        needs_layout_passes=False, use_tc_tiling_on_sc=False),
    scratch_types=[
        pltpu.VMEM((N_PH,), jnp.float32),
        pltpu.VMEM((N_MG,), jnp.float32),
        pltpu.VMEM((N_PH * L,), jnp.float32),
        pltpu.VMEM((N_MG * L,), jnp.float32),
        pltpu.VMEM((CHUNK,), jnp.int32),
        pltpu.VMEM((CHUNK,), jnp.int32),
        pltpu.VMEM((CHUNK,), jnp.int32),
        pltpu.VMEM((CHUNK,), jnp.int32),
        pltpu.VMEM((CHUNK,), jnp.float32),
        pltpu.VMEM((CHUNK,), jnp.float32),
        pltpu.SemaphoreType.DMA,
        pltpu.SemaphoreType.DMA,
        pltpu.SemaphoreType.DMA,
        pltpu.SemaphoreType.DMA,
    ],
)
def _sc_lookup(pi_hbm, mi_hbm, pct_hbm, met_hbm, out_hbm,
               pct_v, met_v, pct_rep, met_rep, pi0, pi1, mi0, mi1, o0, o1,
               sem_in0, sem_in1, sem_out0, sem_out1):
    wid = lax.axis_index("s") * NC + lax.axis_index("c")
    base = wid * PER_W

    pi_bufs = (pi0, pi1)
    mi_bufs = (mi0, mi1)
    o_bufs = (o0, o1)
    sems_in = (sem_in0, sem_in1)
    sems_out = (sem_out0, sem_out1)

    pltpu.sync_copy(pct_hbm, pct_v)
    pltpu.sync_copy(met_hbm, met_v)

    lanes = lax.iota(jnp.int32, L)

    def build_rep(src_ref, rep_ref, n):
        def body(i, _):
            v = plsc.load_gather(src_ref, [jnp.full((L,), i, jnp.int32)])
            rep_ref[pl.ds(i * L, L)] = v
            return 0
        lax.fori_loop(0, n, body, 0)

    build_rep(pct_v, pct_rep, N_PH)
    build_rep(met_v, met_rep, N_MG)

    def start_in(g, b):
        off = base + g * CHUNK
        pltpu.async_copy(pi_hbm.at[pl.ds(off, CHUNK)], pi_bufs[b], sems_in[b])
        pltpu.async_copy(mi_hbm.at[pl.ds(off, CHUNK)], mi_bufs[b], sems_in[b])

    def wait_in(b):
        pltpu.make_async_copy(pi_hbm.at[pl.ds(0, CHUNK)], pi_bufs[b],
                              sems_in[b]).wait()
        pltpu.make_async_copy(mi_hbm.at[pl.ds(0, CHUNK)], mi_bufs[b],
                              sems_in[b]).wait()

    def start_out(g, b):
        off = base + g * CHUNK
        pltpu.async_copy(o_bufs[b], out_hbm.at[pl.ds(off, CHUNK)], sems_out[b])

    def wait_out(b):
        pltpu.make_async_copy(o_bufs[b], out_hbm.at[pl.ds(0, CHUNK)],
                              sems_out[b]).wait()

    def compute(b):
        pi_buf, mi_buf, o_buf = pi_bufs[b], mi_bufs[b], o_bufs[b]

        @plsc.parallel_loop(0, CHUNK, L, unroll=UNROLL)
        def _(off):
            pidx = pi_buf[pl.ds(off, L)]
            midx = mi_buf[pl.ds(off, L)]
            pidx = lax.max(jnp.int32(0), lax.min(pidx, jnp.int32(N_PH - 1)))
            midx = lax.max(jnp.int32(0), lax.min(midx, jnp.int32(N_MG - 1)))
            cv = plsc.load_gather(pct_rep, [(pidx << 4) | lanes])
            mv = plsc.load_gather(met_rep, [(midx << 4) | lanes])
            o_buf[pl.ds(off, L)] = cv * mv

    start_in(0, 0)
    start_in(1, 1)
    for g in range(N_CHUNKS):
        b = g & 1
        wait_in(b)
        if g >= 2:
            wait_out(b)
        compute(b)
        start_out(g, b)
        if g + 2 < N_CHUNKS:
            start_in(g + 2, b)
    wait_out(0)
    wait_out(1)


def kernel(phase_indices, mag_indices, phase_cos_table, mag_exp_table):
    pi = phase_indices.astype(jnp.int32)
    mi = mag_indices.astype(jnp.int32)
    pct = phase_cos_table.astype(jnp.float32)
    met = mag_exp_table.astype(jnp.float32)
    return _sc_lookup(pi, mi, pct, met)


# speedup vs baseline: 877.9744x; 1.0638x over previous
import functools

import jax
import jax.numpy as jnp
from jax import lax
from jax.experimental import pallas as pl
from jax.experimental.pallas import tpu as pltpu
from jax.experimental.pallas import tpu_sc as plsc

D = 8388608
N_PH = 64
N_MG = 1024

NC = 2
NS = 16
L = 16
NW = NC * NS
PER_W = D // NW
CHUNK = 16384
N_CHUNKS = PER_W // CHUNK
UNROLL = 16
GROUPS = CHUNK // (L * UNROLL)

_mesh = plsc.VectorSubcoreMesh(core_axis_name="c", subcore_axis_name="s")


@functools.partial(
    pl.kernel,
    mesh=_mesh,
    out_type=jax.ShapeDtypeStruct((D,), jnp.float32),
    compiler_params=pltpu.CompilerParams(
        needs_layout_passes=False, use_tc_tiling_on_sc=False),
    scratch_types=[
        pltpu.VMEM((N_PH,), jnp.float32),
        pltpu.VMEM((N_MG,), jnp.float32),
        pltpu.VMEM((N_PH * L,), jnp.float32),
        pltpu.VMEM((N_MG * L,), jnp.float32),
        pltpu.VMEM((CHUNK,), jnp.int32),
        pltpu.VMEM((CHUNK,), jnp.int32),
        pltpu.VMEM((CHUNK,), jnp.int32),
        pltpu.VMEM((CHUNK,), jnp.int32),
        pltpu.VMEM((CHUNK,), jnp.float32),
        pltpu.VMEM((CHUNK,), jnp.float32),
        pltpu.SemaphoreType.DMA,
        pltpu.SemaphoreType.DMA,
        pltpu.SemaphoreType.DMA,
        pltpu.SemaphoreType.DMA,
    ],
)
def _sc_lookup(pi_hbm, mi_hbm, pct_hbm, met_hbm, out_hbm,
               pct_v, met_v, pct_rep, met_rep, pi0, pi1, mi0, mi1, o0, o1,
               sem_in0, sem_in1, sem_out0, sem_out1):
    wid = lax.axis_index("s") * NC + lax.axis_index("c")
    base = wid * PER_W

    pi_bufs = (pi0, pi1)
    mi_bufs = (mi0, mi1)
    o_bufs = (o0, o1)
    sems_in = (sem_in0, sem_in1)
    sems_out = (sem_out0, sem_out1)

    pltpu.sync_copy(pct_hbm, pct_v)
    pltpu.sync_copy(met_hbm, met_v)

    lanes = lax.iota(jnp.int32, L)

    def build_rep(src_ref, rep_ref, n):
        def body(i, _):
            v = plsc.load_gather(src_ref, [jnp.full((L,), i, jnp.int32)])
            rep_ref[pl.ds(i * L, L)] = v
            return 0
        lax.fori_loop(0, n, body, 0)

    build_rep(pct_v, pct_rep, N_PH)
    build_rep(met_v, met_rep, N_MG)

    def start_in(g, b):
        off = base + g * CHUNK
        pltpu.async_copy(pi_hbm.at[pl.ds(off, CHUNK)], pi_bufs[b], sems_in[b])
        pltpu.async_copy(mi_hbm.at[pl.ds(off, CHUNK)], mi_bufs[b], sems_in[b])

    def wait_in(b):
        pltpu.make_async_copy(pi_hbm.at[pl.ds(0, CHUNK)], pi_bufs[b],
                              sems_in[b]).wait()
        pltpu.make_async_copy(mi_hbm.at[pl.ds(0, CHUNK)], mi_bufs[b],
                              sems_in[b]).wait()

    def start_out(g, b):
        off = base + g * CHUNK
        pltpu.async_copy(o_bufs[b], out_hbm.at[pl.ds(off, CHUNK)], sems_out[b])

    def wait_out(b):
        pltpu.make_async_copy(o_bufs[b], out_hbm.at[pl.ds(0, CHUNK)],
                              sems_out[b]).wait()

    def compute(b):
        pi_buf, mi_buf, o_buf = pi_bufs[b], mi_bufs[b], o_bufs[b]

        @plsc.parallel_loop(0, CHUNK, L, unroll=UNROLL)
        def _(off):
            pidx = pi_buf[pl.ds(off, L)]
            midx = mi_buf[pl.ds(off, L)]
            pidx = lax.max(jnp.int32(0), lax.min(pidx, jnp.int32(N_PH - 1)))
            midx = lax.max(jnp.int32(0), lax.min(midx, jnp.int32(N_MG - 1)))
            cv = plsc.load_gather(pct_rep, [(pidx << 4) | lanes])
            mv = plsc.load_gather(met_rep, [(midx << 4) | lanes])
            o_buf[pl.ds(off, L)] = cv * mv

    start_in(0, 0)
    start_in(1, 1)
    for g in range(N_CHUNKS):
        b = g & 1
        wait_in(b)
        if g >= 2:
            wait_out(b)
        compute(b)
        start_out(g, b)
        if g + 2 < N_CHUNKS:
            start_in(g + 2, b)
    wait_out(0)
    wait_out(1)


def kernel(phase_indices, mag_indices, phase_cos_table, mag_exp_table):
    pi = phase_indices.astype(jnp.int32)
    mi = mag_indices.astype(jnp.int32)
    pct = phase_cos_table.astype(jnp.float32)
    met = mag_exp_table.astype(jnp.float32)
    return _sc_lookup(pi, mi, pct, met)
